# Initial kernel scaffold; baseline (speedup 1.0000x reference)
import functools

import jax
import jax.numpy as jnp
from jax import lax
from jax.experimental import pallas as pl
from jax.experimental.pallas import tpu as pltpu
from jax.experimental.pallas import tpu_sc as plsc

N = 10000
E = 320000
D = 128
DE = 16
NC, NS, L = 2, 16, 16
NW = NC * NS
EPT = E // NW
C = 80
NCHUNK = EPT // C
RPS = N // NS

_f32 = jnp.float32


def _tc_prep_body(x_ref, wm_ref, bm_ref, ms_ref, md_ref):
    x = x_ref[...]
    ms_ref[...] = jnp.dot(x, wm_ref[0:D, :], preferred_element_type=_f32)
    md_ref[...] = (jnp.dot(x, wm_ref[D:2 * D, :], preferred_element_type=_f32)
                   + bm_ref[...])


def _tc_prep(x, w_msg, b_msg):
    bn = 2000
    return pl.pallas_call(
        _tc_prep_body,
        grid=(N // bn,),
        in_specs=[
            pl.BlockSpec((bn, D), lambda i: (i, 0)),
            pl.BlockSpec((2 * D, D), lambda i: (0, 0)),
            pl.BlockSpec((1, D), lambda i: (0, 0)),
        ],
        out_specs=[
            pl.BlockSpec((bn, D), lambda i: (i, 0)),
            pl.BlockSpec((bn, D), lambda i: (i, 0)),
        ],
        out_shape=[
            jax.ShapeDtypeStruct((N, D), _f32),
            jax.ShapeDtypeStruct((N, D), _f32),
        ],
    )(x, w_msg, b_msg.reshape(1, D))


def _sc_nodeagg_body(ms_hbm, md_hbm, src_hbm, dst_hbm, zeros_agg_hbm,
                     zeros_deg_hbm, onecol_hbm,
                     agg_out, deg_out,
                     sidx_v, didx_v, gs_v, gd_v, m_v, one_v,
                     agg_sh, deg_sh, sem_a, sem_b):
    c = lax.axis_index("c")
    s = lax.axis_index("s")
    wid = c * NS + s
    base = wid * EPT

    pltpu.sync_copy(zeros_agg_hbm.at[pl.ds(s * RPS, RPS)],
                    agg_sh.at[pl.ds(s * RPS, RPS)])
    pltpu.sync_copy(zeros_deg_hbm.at[pl.ds(s * RPS, RPS)],
                    deg_sh.at[pl.ds(s * RPS, RPS)])
    pltpu.sync_copy(onecol_hbm, one_v)
    plsc.subcore_barrier()

    def chunk_body(k, carry):
        cb = base + k * C
        pltpu.sync_copy(src_hbm.at[pl.ds(cb, C)], sidx_v)
        pltpu.sync_copy(dst_hbm.at[pl.ds(cb, C)], didx_v)
        cp1 = pltpu.async_copy(ms_hbm.at[sidx_v], gs_v, sem_a)
        cp2 = pltpu.async_copy(md_hbm.at[didx_v], gd_v, sem_b)
        cp1.wait()
        cp2.wait()

        def edge_body(e, _):
            for j in range(D // L):
                sl = pl.ds(j * L, L)
                m_v[e, sl] = jnp.maximum(gs_v[e, sl] + gd_v[e, sl], 0.0)
            return 0

        lax.fori_loop(0, C, edge_body, 0)
        pltpu.sync_copy(m_v, agg_sh.at[didx_v], add=True)
        pltpu.sync_copy(one_v, deg_sh.at[didx_v], add=True)
        return carry

    lax.fori_loop(0, NCHUNK, chunk_body, 0)
    plsc.subcore_barrier()

    row0 = c * N + s * RPS
    pltpu.sync_copy(agg_sh.at[pl.ds(s * RPS, RPS)],
                    agg_out.at[pl.ds(row0, RPS)])
    pltpu.sync_copy(deg_sh.at[pl.ds(s * RPS, RPS)],
                    deg_out.at[pl.ds(row0, RPS)])


def _sc_nodeagg(ms, md, src, dst):
    mesh = plsc.VectorSubcoreMesh(core_axis_name="c", subcore_axis_name="s",
                                  num_cores=NC, num_subcores=NS)
    zeros_agg = jnp.zeros((N, D), _f32)
    zeros_deg = jnp.zeros((N, DE), _f32)
    onecol = jnp.zeros((C, DE), _f32).at[:, 0].set(1.0)
    fn = functools.partial(
        pl.kernel,
        out_type=[
            jax.ShapeDtypeStruct((NC * N, D), _f32),
            jax.ShapeDtypeStruct((NC * N, DE), _f32),
        ],
        mesh=mesh,
        scratch_types=[
            pltpu.VMEM((C,), jnp.int32),
            pltpu.VMEM((C,), jnp.int32),
            pltpu.VMEM((C, D), _f32),
            pltpu.VMEM((C, D), _f32),
            pltpu.VMEM((C, D), _f32),
            pltpu.VMEM((C, DE), _f32),
            pltpu.VMEM_SHARED((N, D), _f32),
            pltpu.VMEM_SHARED((N, DE), _f32),
            pltpu.SemaphoreType.DMA,
            pltpu.SemaphoreType.DMA,
        ],
    )(_sc_nodeagg_body)
    return fn(ms, md, src, dst, zeros_agg, zeros_deg, onecol)


def _tc_mid_body(x_ref, a0_ref, a1_ref, d0_ref, d1_ref, wn_ref, bn_ref,
                 we1_ref, be1_ref, a_out, b_out):
    agg = a0_ref[...] + a1_ref[...]
    deg = jnp.sum(d0_ref[...] + d1_ref[...], axis=1, keepdims=True)
    aggn = agg / jnp.maximum(deg, 1.0)
    x2 = jnp.maximum(
        jnp.dot(x_ref[...], wn_ref[0:D, :], preferred_element_type=_f32)
        + jnp.dot(aggn, wn_ref[D:2 * D, :], preferred_element_type=_f32)
        + bn_ref[...], 0.0)
    a_out[...] = jnp.dot(x2, we1_ref[0:D, :], preferred_element_type=_f32)
    b_out[...] = (jnp.dot(x2, we1_ref[D:2 * D, :], preferred_element_type=_f32)
                  + be1_ref[...])


def _tc_mid(x, agg0, agg1, deg0, deg1, w_nout, b_nout, w_e1, b_e1):
    bn = 2000
    return pl.pallas_call(
        _tc_mid_body,
        grid=(N // bn,),
        in_specs=[
            pl.BlockSpec((bn, D), lambda i: (i, 0)),
            pl.BlockSpec((bn, D), lambda i: (i, 0)),
            pl.BlockSpec((bn, D), lambda i: (i, 0)),
            pl.BlockSpec((bn, DE), lambda i: (i, 0)),
            pl.BlockSpec((bn, DE), lambda i: (i, 0)),
            pl.BlockSpec((2 * D, D), lambda i: (0, 0)),
            pl.BlockSpec((1, D), lambda i: (0, 0)),
            pl.BlockSpec((2 * D, D), lambda i: (0, 0)),
            pl.BlockSpec((1, D), lambda i: (0, 0)),
        ],
        out_specs=[
            pl.BlockSpec((bn, D), lambda i: (i, 0)),
            pl.BlockSpec((bn, D), lambda i: (i, 0)),
        ],
        out_shape=[
            jax.ShapeDtypeStruct((N, D), _f32),
            jax.ShapeDtypeStruct((N, D), _f32),
        ],
    )(x, agg0, agg1, deg0, deg1, w_nout, b_nout.reshape(1, D),
      w_e1, b_e1.reshape(1, D))


def _sc_edge_body(a_hbm, b_hbm, src_hbm, dst_hbm, w2_hbm,
                  dot_out, ssq_out,
                  sidx_v, didx_v, ga_v, gb_v, dot_v, w2_v, ssq_v,
                  sem_a, sem_b):
    c = lax.axis_index("c")
    s = lax.axis_index("s")
    wid = c * NS + s
    base = wid * EPT

    pltpu.sync_copy(w2_hbm, w2_v)

    def chunk_body(k, ssq_acc):
        cb = base + k * C
        pltpu.sync_copy(src_hbm.at[pl.ds(cb, C)], sidx_v)
        pltpu.sync_copy(dst_hbm.at[pl.ds(cb, C)], didx_v)
        cp1 = pltpu.async_copy(a_hbm.at[sidx_v], ga_v, sem_a)
        cp2 = pltpu.async_copy(b_hbm.at[didx_v], gb_v, sem_b)
        cp1.wait()
        cp2.wait()

        def edge_body(e, sacc):
            dot = jnp.zeros((L,), _f32)
            for j in range(D // L):
                sl = pl.ds(j * L, L)
                pair = ga_v[e, sl] + gb_v[e, sl]
                sacc = sacc + pair * pair
                dot = dot + jnp.maximum(pair, 0.0) * w2_v[j, :]
            dot_v[e, :] = dot
            return sacc

        ssq_acc = lax.fori_loop(0, C, edge_body, ssq_acc)
        pltpu.sync_copy(dot_v, dot_out.at[pl.ds(cb, C)])
        return ssq_acc

    ssq = lax.fori_loop(0, NCHUNK, chunk_body, jnp.zeros((L,), _f32))
    ssq_v[0, :] = ssq
    pltpu.sync_copy(ssq_v, ssq_out.at[pl.ds(wid, 1)])


def _sc_edge(a, b, src, dst, w2):
    mesh = plsc.VectorSubcoreMesh(core_axis_name="c", subcore_axis_name="s",
                                  num_cores=NC, num_subcores=NS)
    fn = functools.partial(
        pl.kernel,
        out_type=[
            jax.ShapeDtypeStruct((E, L), _f32),
            jax.ShapeDtypeStruct((NW, L), _f32),
        ],
        mesh=mesh,
        scratch_types=[
            pltpu.VMEM((C,), jnp.int32),
            pltpu.VMEM((C,), jnp.int32),
            pltpu.VMEM((C, D), _f32),
            pltpu.VMEM((C, D), _f32),
            pltpu.VMEM((C, L), _f32),
            pltpu.VMEM((D // L, L), _f32),
            pltpu.VMEM((1, L), _f32),
            pltpu.SemaphoreType.DMA,
            pltpu.SemaphoreType.DMA,
        ],
    )(_sc_edge_body)
    return fn(a, b, src, dst, w2.reshape(D // L, L))


def _tc_final_body(dot16_ref, ef_ref, wtail_ref, be2_ref, ssq_ref,
                   out_ref, loss_ref):
    s = (jnp.sum(dot16_ref[...], axis=1, keepdims=True)
         + jnp.sum(ef_ref[...] * wtail_ref[...], axis=1, keepdims=True)
         + be2_ref[...])
    out_ref[...] = s

    @pl.when(pl.program_id(0) == 0)
    def _():
        loss_ref[...] = jnp.sum(ssq_ref[...]).reshape(1, 1) / (E * D)


def _tc_final(dot16, ef, w_tail, b_e2, ssq):
    be = 8000
    return pl.pallas_call(
        _tc_final_body,
        grid=(E // be,),
        in_specs=[
            pl.BlockSpec((be, L), lambda i: (i, 0)),
            pl.BlockSpec((be, DE), lambda i: (i, 0)),
            pl.BlockSpec((1, DE), lambda i: (0, 0)),
            pl.BlockSpec((1, 1), lambda i: (0, 0)),
            pl.BlockSpec((NW, L), lambda i: (0, 0)),
        ],
        out_specs=[
            pl.BlockSpec((be, 1), lambda i: (i, 0)),
            pl.BlockSpec((1, 1), lambda i: (0, 0)),
        ],
        out_shape=[
            jax.ShapeDtypeStruct((E, 1), _f32),
            jax.ShapeDtypeStruct((1, 1), _f32),
        ],
    )(dot16, ef, w_tail, b_e2, ssq)


def kernel(node_features, edge_features, edge_index, gt_edges,
           W_msg, b_msg, W_nout, b_nout, W_e1, b_e1, W_e2, b_e2):
    src = edge_index[0]
    dst = edge_index[1]

    ms, md = _tc_prep(node_features, W_msg, b_msg)
    aggp, degp = _sc_nodeagg(ms, md, src, dst)
    a, b = _tc_mid(node_features, aggp[:N], aggp[N:], degp[:N], degp[N:],
                   W_nout, b_nout, W_e1, b_e1)
    dot16, ssq = _sc_edge(a, b, src, dst, W_e2[:D, 0])
    w_tail = W_e2[D:D + DE, 0].reshape(1, DE)
    edge_out, loss = _tc_final(dot16, edge_features, w_tail,
                               b_e2.reshape(1, 1), ssq)
    return edge_out, loss.reshape(())


# speedup vs baseline: 3.9114x; 3.9114x over previous
import functools

import jax
import jax.numpy as jnp
from jax import lax
from jax.experimental import pallas as pl
from jax.experimental.pallas import tpu as pltpu
from jax.experimental.pallas import tpu_sc as plsc

N = 10000
E = 320000
D = 128
DE = 16
NC, NS, L = 2, 16, 16
NW = NC * NS
EPT = E // NW
C = 80
NCHUNK = EPT // C
NP = 10240
RPS = NP // NS

_f32 = jnp.float32


def _tc_prep_body(x_ref, wm_ref, bm_ref, ms_ref, md_ref):
    x = x_ref[...]
    ms_ref[...] = jnp.dot(x, wm_ref[0:D, :], preferred_element_type=_f32)
    md_ref[...] = (jnp.dot(x, wm_ref[D:2 * D, :], preferred_element_type=_f32)
                   + bm_ref[...])


def _tc_prep(x, w_msg, b_msg):
    bn = 2000
    return pl.pallas_call(
        _tc_prep_body,
        grid=(N // bn,),
        in_specs=[
            pl.BlockSpec((bn, D), lambda i: (i, 0)),
            pl.BlockSpec((2 * D, D), lambda i: (0, 0)),
            pl.BlockSpec((1, D), lambda i: (0, 0)),
        ],
        out_specs=[
            pl.BlockSpec((bn, D), lambda i: (i, 0)),
            pl.BlockSpec((bn, D), lambda i: (i, 0)),
        ],
        out_shape=[
            jax.ShapeDtypeStruct((N, D), _f32),
            jax.ShapeDtypeStruct((N, D), _f32),
        ],
    )(x, w_msg, b_msg.reshape(1, D))


def _sc_nodeagg_body(ms_hbm, md_hbm, src_hbm, dst_hbm, zeros_agg_hbm,
                     agg_out, deg_out,
                     sidx_v, didx_v, gs_v, gd_v, m_v, deg_loc,
                     agg_sh, sem_a, sem_b):
    c = lax.axis_index("c")
    s = lax.axis_index("s")
    wid = c * NS + s
    base = wid * EPT

    pltpu.sync_copy(zeros_agg_hbm.at[pl.ds(s * RPS, RPS)],
                    agg_sh.at[pl.ds(s * RPS, RPS)])

    def zero_body(r, _):
        deg_loc[pl.ds(r * L, L)] = jnp.zeros((L,), _f32)
        return 0
    lax.fori_loop(0, NP // L, zero_body, 0)
    plsc.subcore_barrier()

    ones16 = jnp.full((L,), 1.0, _f32)

    def chunk_body(k, carry):
        cb = base + k * C
        pltpu.sync_copy(src_hbm.at[pl.ds(cb, C)], sidx_v)
        pltpu.sync_copy(dst_hbm.at[pl.ds(cb, C)], didx_v)
        cp1 = pltpu.async_copy(ms_hbm.at[sidx_v], gs_v, sem_a)
        cp2 = pltpu.async_copy(md_hbm.at[didx_v], gd_v, sem_b)
        cp1.wait()
        cp2.wait()

        def edge_body(e, _):
            for j in range(D // L):
                sl = pl.ds(j * L, L)
                m_v[e, sl] = jnp.maximum(gs_v[e, sl] + gd_v[e, sl], 0.0)
            return 0

        lax.fori_loop(0, C, edge_body, 0)
        pltpu.sync_copy(m_v, agg_sh.at[didx_v], add=True)
        for g in range(C // L):
            plsc.addupdate_scatter(deg_loc, [didx_v[pl.ds(g * L, L)]], ones16)
        return carry

    lax.fori_loop(0, NCHUNK, chunk_body, 0)
    pltpu.sync_copy(deg_loc, deg_out.at[pl.ds(wid * NP, NP)])
    plsc.subcore_barrier()

    pltpu.sync_copy(agg_sh.at[pl.ds(s * RPS, RPS)],
                    agg_out.at[pl.ds(c * NP + s * RPS, RPS)])


def _sc_nodeagg(ms, md, src, dst):
    mesh = plsc.VectorSubcoreMesh(core_axis_name="c", subcore_axis_name="s",
                                  num_cores=NC, num_subcores=NS)
    zeros_agg = jnp.zeros((NP, D), _f32)
    fn = functools.partial(
        pl.kernel,
        out_type=[
            jax.ShapeDtypeStruct((NC * NP, D), _f32),
            jax.ShapeDtypeStruct((NW * NP,), _f32),
        ],
        mesh=mesh,
        scratch_types=[
            pltpu.VMEM((C,), jnp.int32),
            pltpu.VMEM((C,), jnp.int32),
            pltpu.VMEM((C, D), _f32),
            pltpu.VMEM((C, D), _f32),
            pltpu.VMEM((C, D), _f32),
            pltpu.VMEM((NP,), _f32),
            pltpu.VMEM_SHARED((NP, D), _f32),
            pltpu.SemaphoreType.DMA,
            pltpu.SemaphoreType.DMA,
        ],
        compiler_params=pltpu.CompilerParams(needs_layout_passes=False),
    )(_sc_nodeagg_body)
    return fn(ms, md, src, dst, zeros_agg)


def _tc_degsum_body(dp_ref, out_ref):
    out_ref[...] = jnp.sum(dp_ref[...], axis=0, keepdims=True)


def _tc_degsum(degp):
    bn = 2048
    return pl.pallas_call(
        _tc_degsum_body,
        grid=(NP // bn,),
        in_specs=[pl.BlockSpec((NW, bn), lambda i: (0, i))],
        out_specs=pl.BlockSpec((1, bn), lambda i: (0, i)),
        out_shape=jax.ShapeDtypeStruct((1, NP), _f32),
    )(degp.reshape(NW, NP))


def _tc_mid_body(x_ref, a0_ref, a1_ref, d_ref, wn_ref, bn_ref,
                 we1_ref, be1_ref, a_out, b_out):
    agg = a0_ref[...] + a1_ref[...]
    aggn = agg / jnp.maximum(d_ref[...], 1.0)
    x2 = jnp.maximum(
        jnp.dot(x_ref[...], wn_ref[0:D, :], preferred_element_type=_f32)
        + jnp.dot(aggn, wn_ref[D:2 * D, :], preferred_element_type=_f32)
        + bn_ref[...], 0.0)
    a_out[...] = jnp.dot(x2, we1_ref[0:D, :], preferred_element_type=_f32)
    b_out[...] = (jnp.dot(x2, we1_ref[D:2 * D, :], preferred_element_type=_f32)
                  + be1_ref[...])


def _tc_mid(x, agg0, agg1, deg, w_nout, b_nout, w_e1, b_e1):
    bn = 2000
    return pl.pallas_call(
        _tc_mid_body,
        grid=(N // bn,),
        in_specs=[
            pl.BlockSpec((bn, D), lambda i: (i, 0)),
            pl.BlockSpec((bn, D), lambda i: (i, 0)),
            pl.BlockSpec((bn, D), lambda i: (i, 0)),
            pl.BlockSpec((bn, 1), lambda i: (i, 0)),
            pl.BlockSpec((2 * D, D), lambda i: (0, 0)),
            pl.BlockSpec((1, D), lambda i: (0, 0)),
            pl.BlockSpec((2 * D, D), lambda i: (0, 0)),
            pl.BlockSpec((1, D), lambda i: (0, 0)),
        ],
        out_specs=[
            pl.BlockSpec((bn, D), lambda i: (i, 0)),
            pl.BlockSpec((bn, D), lambda i: (i, 0)),
        ],
        out_shape=[
            jax.ShapeDtypeStruct((N, D), _f32),
            jax.ShapeDtypeStruct((N, D), _f32),
        ],
    )(x, agg0, agg1, deg, w_nout, b_nout.reshape(1, D),
      w_e1, b_e1.reshape(1, D))


def _sc_edge_body(a_hbm, b_hbm, src_hbm, dst_hbm, w2_hbm,
                  dot_out, ssq_out,
                  sidx_v, didx_v, ga_v, gb_v, dot_v, w2_v, ssq_v,
                  sem_a, sem_b):
    c = lax.axis_index("c")
    s = lax.axis_index("s")
    wid = c * NS + s
    base = wid * EPT

    pltpu.sync_copy(w2_hbm, w2_v)

    def chunk_body(k, ssq_acc):
        cb = base + k * C
        pltpu.sync_copy(src_hbm.at[pl.ds(cb, C)], sidx_v)
        pltpu.sync_copy(dst_hbm.at[pl.ds(cb, C)], didx_v)
        cp1 = pltpu.async_copy(a_hbm.at[sidx_v], ga_v, sem_a)
        cp2 = pltpu.async_copy(b_hbm.at[didx_v], gb_v, sem_b)
        cp1.wait()
        cp2.wait()

        def edge_body(e, sacc):
            dot = jnp.zeros((L,), _f32)
            for j in range(D // L):
                sl = pl.ds(j * L, L)
                pair = ga_v[e, sl] + gb_v[e, sl]
                sacc = sacc + pair * pair
                dot = dot + jnp.maximum(pair, 0.0) * w2_v[j, :]
            dot_v[e, :] = dot
            return sacc

        ssq_acc = lax.fori_loop(0, C, edge_body, ssq_acc)
        pltpu.sync_copy(dot_v, dot_out.at[pl.ds(cb, C)])
        return ssq_acc

    ssq = lax.fori_loop(0, NCHUNK, chunk_body, jnp.zeros((L,), _f32))
    for r in range(8):
        ssq_v[r, :] = jnp.zeros((L,), _f32)
    ssq_v[0, :] = ssq
    pltpu.sync_copy(ssq_v, ssq_out.at[wid])


def _sc_edge(a, b, src, dst, w2):
    mesh = plsc.VectorSubcoreMesh(core_axis_name="c", subcore_axis_name="s",
                                  num_cores=NC, num_subcores=NS)
    fn = functools.partial(
        pl.kernel,
        out_type=[
            jax.ShapeDtypeStruct((E, L), _f32),
            jax.ShapeDtypeStruct((NW, 8, L), _f32),
        ],
        mesh=mesh,
        scratch_types=[
            pltpu.VMEM((C,), jnp.int32),
            pltpu.VMEM((C,), jnp.int32),
            pltpu.VMEM((C, D), _f32),
            pltpu.VMEM((C, D), _f32),
            pltpu.VMEM((C, L), _f32),
            pltpu.VMEM((D // L, L), _f32),
            pltpu.VMEM((8, L), _f32),
            pltpu.SemaphoreType.DMA,
            pltpu.SemaphoreType.DMA,
        ],
        compiler_params=pltpu.CompilerParams(needs_layout_passes=False),
    )(_sc_edge_body)
    return fn(a, b, src, dst, w2.reshape(D // L, L))


def _tc_final_body(dot16_ref, ef_ref, wtail_ref, be2_ref, ssq_ref,
                   out_ref, loss_ref):
    s = (jnp.sum(dot16_ref[...], axis=1, keepdims=True)
         + jnp.sum(ef_ref[...] * wtail_ref[...], axis=1, keepdims=True)
         + be2_ref[...])
    out_ref[...] = s

    @pl.when(pl.program_id(0) == 0)
    def _():
        loss_ref[...] = jnp.sum(ssq_ref[...]).reshape(1, 1) / (E * D)


def _tc_final(dot16, ef, w_tail, b_e2, ssq):
    be = 8000
    return pl.pallas_call(
        _tc_final_body,
        grid=(E // be,),
        in_specs=[
            pl.BlockSpec((be, L), lambda i: (i, 0)),
            pl.BlockSpec((be, DE), lambda i: (i, 0)),
            pl.BlockSpec((1, DE), lambda i: (0, 0)),
            pl.BlockSpec((1, 1), lambda i: (0, 0)),
            pl.BlockSpec((NW, 8, L), lambda i: (0, 0, 0)),
        ],
        out_specs=[
            pl.BlockSpec((be, 1), lambda i: (i, 0)),
            pl.BlockSpec((1, 1), lambda i: (0, 0)),
        ],
        out_shape=[
            jax.ShapeDtypeStruct((E, 1), _f32),
            jax.ShapeDtypeStruct((1, 1), _f32),
        ],
    )(dot16, ef, w_tail, b_e2, ssq)


def kernel(node_features, edge_features, edge_index, gt_edges,
           W_msg, b_msg, W_nout, b_nout, W_e1, b_e1, W_e2, b_e2):
    src = edge_index[0]
    dst = edge_index[1]

    ms, md = _tc_prep(node_features, W_msg, b_msg)
    aggp, degp = _sc_nodeagg(ms, md, src, dst)
    deg = _tc_degsum(degp).reshape(NP)[:N].reshape(N, 1)
    a, b = _tc_mid(node_features, aggp[:N], aggp[NP:NP + N], deg,
                   W_nout, b_nout, W_e1, b_e1)
    dot16, ssq = _sc_edge(a, b, src, dst, W_e2[:D, 0])
    w_tail = W_e2[D:D + DE, 0].reshape(1, DE)
    edge_out, loss = _tc_final(dot16, edge_features, w_tail,
                               b_e2.reshape(1, 1), ssq)
    return edge_out, loss.reshape(())


# speedup vs baseline: 3.9724x; 1.0156x over previous
import functools

import jax
import jax.numpy as jnp
from jax import lax
from jax.experimental import pallas as pl
from jax.experimental.pallas import tpu as pltpu
from jax.experimental.pallas import tpu_sc as plsc

N = 10000
E = 320000
D = 128
DE = 16
NC, NS, L = 2, 16, 16
NW = NC * NS
EPT = E // NW
C = 80
NCHUNK = EPT // C
NP = 10240
RPS = NP // NS

_f32 = jnp.float32


def _tc_prep_body(x_ref, wm_ref, bm_ref, ms_ref, md_ref):
    x = x_ref[...]
    ms_ref[...] = jnp.dot(x, wm_ref[0:D, :], preferred_element_type=_f32)
    md_ref[...] = (jnp.dot(x, wm_ref[D:2 * D, :], preferred_element_type=_f32)
                   + bm_ref[...])


def _tc_prep(x, w_msg, b_msg):
    bn = 2000
    return pl.pallas_call(
        _tc_prep_body,
        grid=(N // bn,),
        in_specs=[
            pl.BlockSpec((bn, D), lambda i: (i, 0)),
            pl.BlockSpec((2 * D, D), lambda i: (0, 0)),
            pl.BlockSpec((1, D), lambda i: (0, 0)),
        ],
        out_specs=[
            pl.BlockSpec((bn, D), lambda i: (i, 0)),
            pl.BlockSpec((bn, D), lambda i: (i, 0)),
        ],
        out_shape=[
            jax.ShapeDtypeStruct((N, D), _f32),
            jax.ShapeDtypeStruct((N, D), _f32),
        ],
    )(x, w_msg, b_msg.reshape(1, D))


def _sc_nodeagg_body(ms_hbm, md_hbm, src_hbm, dst_hbm, zeros_agg_hbm,
                     agg_out, deg_out,
                     sidx_v, didx_v, gs_v, gd_v, m_v, deg_loc,
                     agg_sh, sem_a, sem_b):
    c = lax.axis_index("c")
    s = lax.axis_index("s")
    wid = c * NS + s
    base = wid * EPT

    pltpu.sync_copy(zeros_agg_hbm.at[pl.ds(s * RPS, RPS)],
                    agg_sh.at[pl.ds(s * RPS, RPS)])

    def zero_body(r, _):
        deg_loc[pl.ds(r * L, L)] = jnp.zeros((L,), _f32)
        return 0
    lax.fori_loop(0, NP // L, zero_body, 0)
    plsc.subcore_barrier()

    ones16 = jnp.full((L,), 1.0, _f32)

    def chunk_body(k, carry):
        cb = base + k * C
        pltpu.sync_copy(src_hbm.at[pl.ds(cb, C)], sidx_v)
        pltpu.sync_copy(dst_hbm.at[pl.ds(cb, C)], didx_v)
        cp1 = pltpu.async_copy(ms_hbm.at[sidx_v], gs_v, sem_a)
        cp2 = pltpu.async_copy(md_hbm.at[didx_v], gd_v, sem_b)
        cp1.wait()
        cp2.wait()

        @plsc.parallel_loop(0, C, unroll=4)
        def _(e):
            for j in range(D // L):
                sl = pl.ds(j * L, L)
                m_v[e, sl] = jnp.maximum(gs_v[e, sl] + gd_v[e, sl], 0.0)
        pltpu.sync_copy(m_v, agg_sh.at[didx_v], add=True)
        for g in range(C // L):
            plsc.addupdate_scatter(deg_loc, [didx_v[pl.ds(g * L, L)]], ones16)
        return carry

    lax.fori_loop(0, NCHUNK, chunk_body, 0)
    pltpu.sync_copy(deg_loc, deg_out.at[pl.ds(wid * NP, NP)])
    plsc.subcore_barrier()

    pltpu.sync_copy(agg_sh.at[pl.ds(s * RPS, RPS)],
                    agg_out.at[pl.ds(c * NP + s * RPS, RPS)])


def _sc_nodeagg(ms, md, src, dst):
    mesh = plsc.VectorSubcoreMesh(core_axis_name="c", subcore_axis_name="s",
                                  num_cores=NC, num_subcores=NS)
    zeros_agg = jnp.zeros((NP, D), _f32)
    fn = functools.partial(
        pl.kernel,
        out_type=[
            jax.ShapeDtypeStruct((NC * NP, D), _f32),
            jax.ShapeDtypeStruct((NW * NP,), _f32),
        ],
        mesh=mesh,
        scratch_types=[
            pltpu.VMEM((C,), jnp.int32),
            pltpu.VMEM((C,), jnp.int32),
            pltpu.VMEM((C, D), _f32),
            pltpu.VMEM((C, D), _f32),
            pltpu.VMEM((C, D), _f32),
            pltpu.VMEM((NP,), _f32),
            pltpu.VMEM_SHARED((NP, D), _f32),
            pltpu.SemaphoreType.DMA,
            pltpu.SemaphoreType.DMA,
        ],
        compiler_params=pltpu.CompilerParams(needs_layout_passes=False),
    )(_sc_nodeagg_body)
    return fn(ms, md, src, dst, zeros_agg)


def _tc_degsum_body(dp_ref, out_ref):
    out_ref[...] = jnp.sum(dp_ref[...], axis=0, keepdims=True)


def _tc_degsum(degp):
    bn = 2048
    return pl.pallas_call(
        _tc_degsum_body,
        grid=(NP // bn,),
        in_specs=[pl.BlockSpec((NW, bn), lambda i: (0, i))],
        out_specs=pl.BlockSpec((1, bn), lambda i: (0, i)),
        out_shape=jax.ShapeDtypeStruct((1, NP), _f32),
    )(degp.reshape(NW, NP))


def _tc_mid_body(x_ref, a0_ref, a1_ref, d_ref, wn_ref, bn_ref,
                 we1_ref, be1_ref, a_out, b_out):
    agg = a0_ref[...] + a1_ref[...]
    aggn = agg / jnp.maximum(d_ref[...], 1.0)
    x2 = jnp.maximum(
        jnp.dot(x_ref[...], wn_ref[0:D, :], preferred_element_type=_f32)
        + jnp.dot(aggn, wn_ref[D:2 * D, :], preferred_element_type=_f32)
        + bn_ref[...], 0.0)
    a_out[...] = jnp.dot(x2, we1_ref[0:D, :], preferred_element_type=_f32)
    b_out[...] = (jnp.dot(x2, we1_ref[D:2 * D, :], preferred_element_type=_f32)
                  + be1_ref[...])


def _tc_mid(x, agg0, agg1, deg, w_nout, b_nout, w_e1, b_e1):
    bn = 2000
    return pl.pallas_call(
        _tc_mid_body,
        grid=(N // bn,),
        in_specs=[
            pl.BlockSpec((bn, D), lambda i: (i, 0)),
            pl.BlockSpec((bn, D), lambda i: (i, 0)),
            pl.BlockSpec((bn, D), lambda i: (i, 0)),
            pl.BlockSpec((bn, 1), lambda i: (i, 0)),
            pl.BlockSpec((2 * D, D), lambda i: (0, 0)),
            pl.BlockSpec((1, D), lambda i: (0, 0)),
            pl.BlockSpec((2 * D, D), lambda i: (0, 0)),
            pl.BlockSpec((1, D), lambda i: (0, 0)),
        ],
        out_specs=[
            pl.BlockSpec((bn, D), lambda i: (i, 0)),
            pl.BlockSpec((bn, D), lambda i: (i, 0)),
        ],
        out_shape=[
            jax.ShapeDtypeStruct((N, D), _f32),
            jax.ShapeDtypeStruct((N, D), _f32),
        ],
    )(x, agg0, agg1, deg, w_nout, b_nout.reshape(1, D),
      w_e1, b_e1.reshape(1, D))


def _sc_edge_body(a_hbm, b_hbm, src_hbm, dst_hbm, w2_hbm,
                  dot_out, ssq_out,
                  sidx_v, didx_v, ga_v, gb_v, dot_v, w2_v, ssq_v,
                  sem_a, sem_b):
    c = lax.axis_index("c")
    s = lax.axis_index("s")
    wid = c * NS + s
    base = wid * EPT

    pltpu.sync_copy(w2_hbm, w2_v)

    def chunk_body(k, ssq_acc):
        cb = base + k * C
        pltpu.sync_copy(src_hbm.at[pl.ds(cb, C)], sidx_v)
        pltpu.sync_copy(dst_hbm.at[pl.ds(cb, C)], didx_v)
        cp1 = pltpu.async_copy(a_hbm.at[sidx_v], ga_v, sem_a)
        cp2 = pltpu.async_copy(b_hbm.at[didx_v], gb_v, sem_b)
        cp1.wait()
        cp2.wait()

        def edge_body(e, sacc):
            dot = jnp.zeros((L,), _f32)
            for j in range(D // L):
                sl = pl.ds(j * L, L)
                pair = ga_v[e, sl] + gb_v[e, sl]
                sacc = sacc + pair * pair
                dot = dot + jnp.maximum(pair, 0.0) * w2_v[j, :]
            dot_v[e, :] = dot
            return sacc

        ssq_acc = plsc.parallel_loop(0, C, unroll=2, carry=ssq_acc)(edge_body)
        pltpu.sync_copy(dot_v, dot_out.at[pl.ds(cb, C)])
        return ssq_acc

    ssq = lax.fori_loop(0, NCHUNK, chunk_body, jnp.zeros((L,), _f32))
    for r in range(8):
        ssq_v[r, :] = jnp.zeros((L,), _f32)
    ssq_v[0, :] = ssq
    pltpu.sync_copy(ssq_v, ssq_out.at[wid])


def _sc_edge(a, b, src, dst, w2):
    mesh = plsc.VectorSubcoreMesh(core_axis_name="c", subcore_axis_name="s",
                                  num_cores=NC, num_subcores=NS)
    fn = functools.partial(
        pl.kernel,
        out_type=[
            jax.ShapeDtypeStruct((E, L), _f32),
            jax.ShapeDtypeStruct((NW, 8, L), _f32),
        ],
        mesh=mesh,
        scratch_types=[
            pltpu.VMEM((C,), jnp.int32),
            pltpu.VMEM((C,), jnp.int32),
            pltpu.VMEM((C, D), _f32),
            pltpu.VMEM((C, D), _f32),
            pltpu.VMEM((C, L), _f32),
            pltpu.VMEM((D // L, L), _f32),
            pltpu.VMEM((8, L), _f32),
            pltpu.SemaphoreType.DMA,
            pltpu.SemaphoreType.DMA,
        ],
        compiler_params=pltpu.CompilerParams(needs_layout_passes=False),
    )(_sc_edge_body)
    return fn(a, b, src, dst, w2.reshape(D // L, L))


def _tc_final_body(dot16_ref, ef_ref, wtail_ref, be2_ref, ssq_ref,
                   out_ref, loss_ref):
    s = (jnp.sum(dot16_ref[...], axis=1, keepdims=True)
         + jnp.sum(ef_ref[...] * wtail_ref[...], axis=1, keepdims=True)
         + be2_ref[...])
    out_ref[...] = s

    @pl.when(pl.program_id(0) == 0)
    def _():
        loss_ref[...] = jnp.sum(ssq_ref[...]).reshape(1, 1) / (E * D)


def _tc_final(dot16, ef, w_tail, b_e2, ssq):
    be = 8000
    return pl.pallas_call(
        _tc_final_body,
        grid=(E // be,),
        in_specs=[
            pl.BlockSpec((be, L), lambda i: (i, 0)),
            pl.BlockSpec((be, DE), lambda i: (i, 0)),
            pl.BlockSpec((1, DE), lambda i: (0, 0)),
            pl.BlockSpec((1, 1), lambda i: (0, 0)),
            pl.BlockSpec((NW, 8, L), lambda i: (0, 0, 0)),
        ],
        out_specs=[
            pl.BlockSpec((be, 1), lambda i: (i, 0)),
            pl.BlockSpec((1, 1), lambda i: (0, 0)),
        ],
        out_shape=[
            jax.ShapeDtypeStruct((E, 1), _f32),
            jax.ShapeDtypeStruct((1, 1), _f32),
        ],
    )(dot16, ef, w_tail, b_e2, ssq)


def kernel(node_features, edge_features, edge_index, gt_edges,
           W_msg, b_msg, W_nout, b_nout, W_e1, b_e1, W_e2, b_e2):
    src = edge_index[0]
    dst = edge_index[1]

    ms, md = _tc_prep(node_features, W_msg, b_msg)
    aggp, degp = _sc_nodeagg(ms, md, src, dst)
    deg = _tc_degsum(degp).reshape(NP)[:N].reshape(N, 1)
    a, b = _tc_mid(node_features, aggp[:N], aggp[NP:NP + N], deg,
                   W_nout, b_nout, W_e1, b_e1)
    dot16, ssq = _sc_edge(a, b, src, dst, W_e2[:D, 0])
    w_tail = W_e2[D:D + DE, 0].reshape(1, DE)
    edge_out, loss = _tc_final(dot16, edge_features, w_tail,
                               b_e2.reshape(1, 1), ssq)
    return edge_out, loss.reshape(())


# speedup vs baseline: 4.4405x; 1.1178x over previous
import functools

import jax
import jax.numpy as jnp
from jax import lax
from jax.experimental import pallas as pl
from jax.experimental.pallas import tpu as pltpu
from jax.experimental.pallas import tpu_sc as plsc

N = 10000
E = 320000
D = 128
DE = 16
NC, NS, L = 2, 16, 16
NW = NC * NS
EPT = E // NW
C1 = 16
NCHUNK1 = EPT // C1
NITER1 = NCHUNK1 // 4
C2 = 80
NCHUNK2 = EPT // C2
NP = 10240
RPS = NP // NS

_f32 = jnp.float32


def _tc_prep_body(x_ref, wm_ref, bm_ref, ms_ref, md_ref):
    x = x_ref[...]
    ms_ref[...] = jnp.dot(x, wm_ref[0:D, :], preferred_element_type=_f32)
    md_ref[...] = (jnp.dot(x, wm_ref[D:2 * D, :], preferred_element_type=_f32)
                   + bm_ref[...])


def _tc_prep(x, w_msg, b_msg):
    bn = 2000
    return pl.pallas_call(
        _tc_prep_body,
        grid=(N // bn,),
        in_specs=[
            pl.BlockSpec((bn, D), lambda i: (i, 0)),
            pl.BlockSpec((2 * D, D), lambda i: (0, 0)),
            pl.BlockSpec((1, D), lambda i: (0, 0)),
        ],
        out_specs=[
            pl.BlockSpec((bn, D), lambda i: (i, 0)),
            pl.BlockSpec((bn, D), lambda i: (i, 0)),
        ],
        out_shape=[
            jax.ShapeDtypeStruct((N, D), _f32),
            jax.ShapeDtypeStruct((N, D), _f32),
        ],
    )(x, w_msg, b_msg.reshape(1, D))


def _sc_nodeagg_body(ms_hbm, md_hbm, src_hbm, dst_hbm, zeros_agg_hbm,
                     agg_out, deg_out,
                     sidx_v, didx_v, gs_v, gd_v, m_v, deg_loc,
                     agg_sh, sem_i, sem_g0, sem_g1, sem_s0, sem_s1):
    c = lax.axis_index("c")
    s = lax.axis_index("s")
    wid = c * NS + s
    sem_g = (sem_g0, sem_g1)
    sem_s = (sem_s0, sem_s1)

    pltpu.sync_copy(zeros_agg_hbm.at[pl.ds(s * RPS, RPS)],
                    agg_sh.at[pl.ds(s * RPS, RPS)])

    def zero_body(r, _):
        deg_loc[pl.ds(r * L, L)] = jnp.zeros((L,), _f32)
        return 0
    lax.fori_loop(0, NP // L, zero_body, 0)
    plsc.subcore_barrier()

    ones16 = jnp.full((L,), 1.0, _f32)

    def fire_idx(k, slot):
        pltpu.async_copy(src_hbm.at[wid, k], sidx_v.at[slot], sem_i)
        pltpu.async_copy(dst_hbm.at[wid, k], didx_v.at[slot], sem_i)

    def wait_idx(k, slot):
        pltpu.make_async_copy(src_hbm.at[wid, k], sidx_v.at[slot],
                              sem_i).wait()
        pltpu.make_async_copy(dst_hbm.at[wid, k], didx_v.at[slot],
                              sem_i).wait()

    def fire_gathers(slot, b):
        pltpu.async_copy(ms_hbm.at[sidx_v.at[slot]], gs_v.at[b], sem_g[b])
        pltpu.async_copy(md_hbm.at[didx_v.at[slot]], gd_v.at[b], sem_g[b])

    def wait_gathers(slot, b):
        pltpu.make_async_copy(ms_hbm.at[sidx_v.at[slot]], gs_v.at[b],
                              sem_g[b]).wait()
        pltpu.make_async_copy(md_hbm.at[didx_v.at[slot]], gd_v.at[b],
                              sem_g[b]).wait()

    def wait_scatter(slot, b):
        pltpu.make_async_copy(m_v.at[b], agg_sh.at[didx_v.at[slot]],
                              sem_s[b]).wait()

    def process(slot, b):
        @plsc.parallel_loop(0, C1, unroll=4)
        def _(e):
            for j in range(D // L):
                sl = pl.ds(j * L, L)
                m_v[b, e, sl] = jnp.maximum(
                    gs_v[b, e, sl] + gd_v[b, e, sl], 0.0)
        for g in range(C1 // L):
            plsc.addupdate_scatter(
                deg_loc, [didx_v[slot, pl.ds(g * L, L)]], ones16)
        pltpu.async_copy(m_v.at[b], agg_sh.at[didx_v.at[slot]], sem_s[b],
                         add=True)

    fire_idx(0, 0)
    fire_idx(1, 1)
    wait_idx(0, 0)
    fire_gathers(0, 0)

    def loop_body(i, carry):
        for b in range(4):
            k = i * 4 + b
            gb = b % 2
            wait_gathers(b, gb)
            wait_idx(k + 1, (b + 1) % 4)
            fire_gathers((b + 1) % 4, 1 - gb)
            if b < 2:
                @pl.when(i >= 1)
                def _():
                    wait_scatter((b + 2) % 4, gb)
            else:
                wait_scatter((b + 2) % 4, gb)
            if b < 3:
                fire_idx(k + 2, (b + 2) % 4)
            else:
                @pl.when(i < NITER1 - 1)
                def _():
                    fire_idx(k + 2, (b + 2) % 4)
            process(b, gb)
        return carry

    lax.fori_loop(0, NITER1, loop_body, 0)
    wait_gathers(0, 0)
    wait_scatter(2, 0)
    process(0, 0)
    wait_scatter(3, 1)
    wait_scatter(0, 0)

    pltpu.sync_copy(deg_loc, deg_out.at[pl.ds(wid * NP, NP)])
    plsc.subcore_barrier()

    pltpu.sync_copy(agg_sh.at[pl.ds(s * RPS, RPS)],
                    agg_out.at[pl.ds(c * NP + s * RPS, RPS)])


def _sc_nodeagg(ms, md, src3, dst3):
    mesh = plsc.VectorSubcoreMesh(core_axis_name="c", subcore_axis_name="s",
                                  num_cores=NC, num_subcores=NS)
    zeros_agg = jnp.zeros((NP, D), _f32)
    fn = functools.partial(
        pl.kernel,
        out_type=[
            jax.ShapeDtypeStruct((NC * NP, D), _f32),
            jax.ShapeDtypeStruct((NW * NP,), _f32),
        ],
        mesh=mesh,
        scratch_types=[
            pltpu.VMEM((4, C1), jnp.int32),
            pltpu.VMEM((4, C1), jnp.int32),
            pltpu.VMEM((2, C1, D), _f32),
            pltpu.VMEM((2, C1, D), _f32),
            pltpu.VMEM((2, C1, D), _f32),
            pltpu.VMEM((NP,), _f32),
            pltpu.VMEM_SHARED((NP, D), _f32),
            pltpu.SemaphoreType.DMA,
            pltpu.SemaphoreType.DMA,
            pltpu.SemaphoreType.DMA,
            pltpu.SemaphoreType.DMA,
            pltpu.SemaphoreType.DMA,
        ],
        compiler_params=pltpu.CompilerParams(needs_layout_passes=False),
    )(_sc_nodeagg_body)
    return fn(ms, md, src3, dst3, zeros_agg)


def _tc_degsum_body(dp_ref, out_ref):
    out_ref[...] = jnp.sum(dp_ref[...], axis=0, keepdims=True)


def _tc_degsum(degp):
    bn = 2048
    return pl.pallas_call(
        _tc_degsum_body,
        grid=(NP // bn,),
        in_specs=[pl.BlockSpec((NW, bn), lambda i: (0, i))],
        out_specs=pl.BlockSpec((1, bn), lambda i: (0, i)),
        out_shape=jax.ShapeDtypeStruct((1, NP), _f32),
    )(degp.reshape(NW, NP))


def _tc_mid_body(x_ref, a0_ref, a1_ref, d_ref, wn_ref, bn_ref,
                 we1_ref, be1_ref, a_out, b_out):
    agg = a0_ref[...] + a1_ref[...]
    aggn = agg / jnp.maximum(d_ref[...], 1.0)
    x2 = jnp.maximum(
        jnp.dot(x_ref[...], wn_ref[0:D, :], preferred_element_type=_f32)
        + jnp.dot(aggn, wn_ref[D:2 * D, :], preferred_element_type=_f32)
        + bn_ref[...], 0.0)
    a_out[...] = jnp.dot(x2, we1_ref[0:D, :], preferred_element_type=_f32)
    b_out[...] = (jnp.dot(x2, we1_ref[D:2 * D, :], preferred_element_type=_f32)
                  + be1_ref[...])


def _tc_mid(x, agg0, agg1, deg, w_nout, b_nout, w_e1, b_e1):
    bn = 2000
    return pl.pallas_call(
        _tc_mid_body,
        grid=(N // bn,),
        in_specs=[
            pl.BlockSpec((bn, D), lambda i: (i, 0)),
            pl.BlockSpec((bn, D), lambda i: (i, 0)),
            pl.BlockSpec((bn, D), lambda i: (i, 0)),
            pl.BlockSpec((bn, 1), lambda i: (i, 0)),
            pl.BlockSpec((2 * D, D), lambda i: (0, 0)),
            pl.BlockSpec((1, D), lambda i: (0, 0)),
            pl.BlockSpec((2 * D, D), lambda i: (0, 0)),
            pl.BlockSpec((1, D), lambda i: (0, 0)),
        ],
        out_specs=[
            pl.BlockSpec((bn, D), lambda i: (i, 0)),
            pl.BlockSpec((bn, D), lambda i: (i, 0)),
        ],
        out_shape=[
            jax.ShapeDtypeStruct((N, D), _f32),
            jax.ShapeDtypeStruct((N, D), _f32),
        ],
    )(x, agg0, agg1, deg, w_nout, b_nout.reshape(1, D),
      w_e1, b_e1.reshape(1, D))


def _sc_edge_body(a_hbm, b_hbm, src_hbm, dst_hbm, w2_hbm,
                  dot_out, ssq_out,
                  sidx_all, didx_all, ga_v, gb_v, dot_v, w2_v, ssq_v,
                  sem_g0, sem_g1, sem_w0, sem_w1):
    c = lax.axis_index("c")
    s = lax.axis_index("s")
    wid = c * NS + s
    base = wid * EPT
    sem_g = (sem_g0, sem_g1)
    sem_w = (sem_w0, sem_w1)

    pltpu.sync_copy(w2_hbm, w2_v)
    pltpu.sync_copy(src_hbm.at[wid], sidx_all)
    pltpu.sync_copy(dst_hbm.at[wid], didx_all)

    def fire_gathers(k, b):
        pltpu.async_copy(a_hbm.at[sidx_all.at[k]], ga_v.at[b], sem_g[b])
        pltpu.async_copy(b_hbm.at[didx_all.at[k]], gb_v.at[b], sem_g[b])

    def wait_gathers(k, b):
        pltpu.make_async_copy(a_hbm.at[sidx_all.at[k]], ga_v.at[b],
                              sem_g[b]).wait()
        pltpu.make_async_copy(b_hbm.at[didx_all.at[k]], gb_v.at[b],
                              sem_g[b]).wait()

    def wait_write(k, b):
        pltpu.make_async_copy(dot_v.at[b],
                              dot_out.at[pl.ds(base + k * C2, C2)],
                              sem_w[b]).wait()

    def process(k, b, sacc0):
        def edge_body(e, sacc):
            dot = jnp.zeros((L,), _f32)
            for j in range(D // L):
                sl = pl.ds(j * L, L)
                pair = ga_v[b, e, sl] + gb_v[b, e, sl]
                sacc = sacc + pair * pair
                dot = dot + jnp.maximum(pair, 0.0) * w2_v[j, :]
            dot_v[b, e, :] = dot
            return sacc

        sacc0 = plsc.parallel_loop(0, C2, unroll=2, carry=sacc0)(edge_body)
        pltpu.async_copy(dot_v.at[b], dot_out.at[pl.ds(base + k * C2, C2)],
                         sem_w[b])
        return sacc0

    fire_gathers(0, 0)

    def loop_body(i, ssq_acc):
        for b in range(2):
            k = i * 2 + b
            wait_gathers(k, b)
            fire_gathers(k + 1, 1 - b)

            @pl.when(i >= 1)
            def _():
                wait_write(k - 2, b)
            ssq_acc = process(k, b, ssq_acc)
        return ssq_acc

    ssq = lax.fori_loop(0, (NCHUNK2 - 1) // 2, loop_body,
                        jnp.zeros((L,), _f32))
    klast = NCHUNK2 - 1
    wait_gathers(klast, 0)
    wait_write(klast - 2, 0)
    ssq = process(klast, 0, ssq)
    wait_write(klast, 0)
    wait_write(klast - 1, 1)

    for r in range(8):
        ssq_v[r, :] = jnp.zeros((L,), _f32)
    ssq_v[0, :] = ssq
    pltpu.sync_copy(ssq_v, ssq_out.at[wid])


def _sc_edge(a, b, src3, dst3, w2):
    mesh = plsc.VectorSubcoreMesh(core_axis_name="c", subcore_axis_name="s",
                                  num_cores=NC, num_subcores=NS)
    fn = functools.partial(
        pl.kernel,
        out_type=[
            jax.ShapeDtypeStruct((E, L), _f32),
            jax.ShapeDtypeStruct((NW, 8, L), _f32),
        ],
        mesh=mesh,
        scratch_types=[
            pltpu.VMEM((NCHUNK2, C2), jnp.int32),
            pltpu.VMEM((NCHUNK2, C2), jnp.int32),
            pltpu.VMEM((2, C2, D), _f32),
            pltpu.VMEM((2, C2, D), _f32),
            pltpu.VMEM((2, C2, L), _f32),
            pltpu.VMEM((D // L, L), _f32),
            pltpu.VMEM((8, L), _f32),
            pltpu.SemaphoreType.DMA,
            pltpu.SemaphoreType.DMA,
            pltpu.SemaphoreType.DMA,
            pltpu.SemaphoreType.DMA,
        ],
        compiler_params=pltpu.CompilerParams(needs_layout_passes=False),
    )(_sc_edge_body)
    return fn(a, b, src3, dst3, w2.reshape(D // L, L))


def _tc_final_body(dot16_ref, ef_ref, wtail_ref, be2_ref, ssq_ref,
                   out_ref, loss_ref):
    s = (jnp.sum(dot16_ref[...], axis=1, keepdims=True)
         + jnp.sum(ef_ref[...] * wtail_ref[...], axis=1, keepdims=True)
         + be2_ref[...])
    out_ref[...] = s

    @pl.when(pl.program_id(0) == 0)
    def _():
        loss_ref[...] = jnp.sum(ssq_ref[...]).reshape(1, 1) / (E * D)


def _tc_final(dot16, ef, w_tail, b_e2, ssq):
    be = 8000
    return pl.pallas_call(
        _tc_final_body,
        grid=(E // be,),
        in_specs=[
            pl.BlockSpec((be, L), lambda i: (i, 0)),
            pl.BlockSpec((be, DE), lambda i: (i, 0)),
            pl.BlockSpec((1, DE), lambda i: (0, 0)),
            pl.BlockSpec((1, 1), lambda i: (0, 0)),
            pl.BlockSpec((NW, 8, L), lambda i: (0, 0, 0)),
        ],
        out_specs=[
            pl.BlockSpec((be, 1), lambda i: (i, 0)),
            pl.BlockSpec((1, 1), lambda i: (0, 0)),
        ],
        out_shape=[
            jax.ShapeDtypeStruct((E, 1), _f32),
            jax.ShapeDtypeStruct((1, 1), _f32),
        ],
    )(dot16, ef, w_tail, b_e2, ssq)


def kernel(node_features, edge_features, edge_index, gt_edges,
           W_msg, b_msg, W_nout, b_nout, W_e1, b_e1, W_e2, b_e2):
    src1 = edge_index[0].reshape(NW, NCHUNK1, C1)
    dst1 = edge_index[1].reshape(NW, NCHUNK1, C1)
    src2 = edge_index[0].reshape(NW, NCHUNK2, C2)
    dst2 = edge_index[1].reshape(NW, NCHUNK2, C2)

    ms, md = _tc_prep(node_features, W_msg, b_msg)
    aggp, degp = _sc_nodeagg(ms, md, src1, dst1)
    deg = _tc_degsum(degp).reshape(NP)[:N].reshape(N, 1)
    a, b = _tc_mid(node_features, aggp[:N], aggp[NP:NP + N], deg,
                   W_nout, b_nout, W_e1, b_e1)
    dot16, ssq = _sc_edge(a, b, src2, dst2, W_e2[:D, 0])
    w_tail = W_e2[D:D + DE, 0].reshape(1, DE)
    edge_out, loss = _tc_final(dot16, edge_features, w_tail,
                               b_e2.reshape(1, 1), ssq)
    return edge_out, loss.reshape(())


# speedup vs baseline: 5.3022x; 1.1941x over previous
import functools

import jax
import jax.numpy as jnp
from jax import lax
from jax.experimental import pallas as pl
from jax.experimental.pallas import tpu as pltpu
from jax.experimental.pallas import tpu_sc as plsc

N = 10000
E = 320000
D = 128
DE = 16
NC, NS, L = 2, 16, 16
NW = NC * NS
EPT = E // NW
EP = 320512
EPT1 = EP // NW
C1 = 32
NCHUNK1 = EPT1 // C1
NITER1 = NCHUNK1 // 4
C2 = 80
NCHUNK2 = EPT // C2
NP = 10240
RPS = NP // NS

_f32 = jnp.float32


def _tc_prep_body(x_ref, wm_ref, bm_ref, ms_ref, md_ref):
    x = x_ref[...]
    ms_ref[...] = jnp.dot(x, wm_ref[0:D, :], preferred_element_type=_f32)
    md_ref[...] = (jnp.dot(x, wm_ref[D:2 * D, :], preferred_element_type=_f32)
                   + bm_ref[...])


def _tc_prep(x, w_msg, b_msg):
    bn = 2048
    xp = jnp.concatenate([x, jnp.zeros((NP - N, D), _f32)])
    return pl.pallas_call(
        _tc_prep_body,
        grid=(NP // bn,),
        in_specs=[
            pl.BlockSpec((bn, D), lambda i: (i, 0)),
            pl.BlockSpec((2 * D, D), lambda i: (0, 0)),
            pl.BlockSpec((1, D), lambda i: (0, 0)),
        ],
        out_specs=[
            pl.BlockSpec((bn, D), lambda i: (i, 0)),
            pl.BlockSpec((bn, D), lambda i: (i, 0)),
        ],
        out_shape=[
            jax.ShapeDtypeStruct((NP, D), _f32),
            jax.ShapeDtypeStruct((NP, D), _f32),
        ],
    )(xp, w_msg, b_msg.reshape(1, D))


def _sc_nodeagg_body(ms_hbm, md_hbm, src_hbm, dst_hbm, zeros_agg_hbm,
                     agg_out, deg_out,
                     sidx_v, didx_v, gs_v, gd_v, m_v, deg_loc,
                     agg_sh, sem_i, sem_g0, sem_g1, sem_s0, sem_s1):
    c = lax.axis_index("c")
    s = lax.axis_index("s")
    wid = c * NS + s
    sem_g = (sem_g0, sem_g1)
    sem_s = (sem_s0, sem_s1)

    pltpu.sync_copy(zeros_agg_hbm.at[pl.ds(s * RPS, RPS)],
                    agg_sh.at[pl.ds(s * RPS, RPS)])

    def zero_body(r, _):
        deg_loc[pl.ds(r * L, L)] = jnp.zeros((L,), _f32)
        return 0
    lax.fori_loop(0, NP // L, zero_body, 0)
    plsc.subcore_barrier()

    ones16 = jnp.full((L,), 1.0, _f32)

    def fire_idx(k, slot):
        pltpu.async_copy(src_hbm.at[wid, k], sidx_v.at[slot], sem_i)
        pltpu.async_copy(dst_hbm.at[wid, k], didx_v.at[slot], sem_i)

    def wait_idx(k, slot):
        pltpu.make_async_copy(src_hbm.at[wid, k], sidx_v.at[slot],
                              sem_i).wait()
        pltpu.make_async_copy(dst_hbm.at[wid, k], didx_v.at[slot],
                              sem_i).wait()

    def fire_gathers(slot, b):
        pltpu.async_copy(ms_hbm.at[sidx_v.at[slot]], gs_v.at[b], sem_g[b])
        pltpu.async_copy(md_hbm.at[didx_v.at[slot]], gd_v.at[b], sem_g[b])

    def wait_gathers(slot, b):
        pltpu.make_async_copy(ms_hbm.at[sidx_v.at[slot]], gs_v.at[b],
                              sem_g[b]).wait()
        pltpu.make_async_copy(md_hbm.at[didx_v.at[slot]], gd_v.at[b],
                              sem_g[b]).wait()

    def wait_scatter(slot, b):
        pltpu.make_async_copy(m_v.at[b], agg_sh.at[didx_v.at[slot]],
                              sem_s[b]).wait()

    def process(slot, b):
        @plsc.parallel_loop(0, C1, unroll=4)
        def _(e):
            for j in range(D // L):
                sl = pl.ds(j * L, L)
                m_v[b, e, sl] = jnp.maximum(
                    gs_v[b, e, sl] + gd_v[b, e, sl], 0.0)
        for g in range(C1 // L):
            plsc.addupdate_scatter(
                deg_loc, [didx_v[slot, pl.ds(g * L, L)]], ones16)
        pltpu.async_copy(m_v.at[b], agg_sh.at[didx_v.at[slot]], sem_s[b],
                         add=True)

    fire_idx(0, 0)
    fire_idx(1, 1)
    wait_idx(0, 0)
    fire_gathers(0, 0)

    def loop_body(i, carry):
        for b in range(4):
            k = i * 4 + b
            gb = b % 2
            wait_gathers(b, gb)
            wait_idx(k + 1, (b + 1) % 4)
            fire_gathers((b + 1) % 4, 1 - gb)
            if b < 2:
                @pl.when(i >= 1)
                def _():
                    wait_scatter((b + 2) % 4, gb)
            else:
                wait_scatter((b + 2) % 4, gb)
            if b < 3:
                fire_idx(k + 2, (b + 2) % 4)
            else:
                @pl.when(i < NITER1 - 1)
                def _():
                    fire_idx(k + 2, (b + 2) % 4)
            process(b, gb)
        return carry

    lax.fori_loop(0, NITER1, loop_body, 0)
    wait_gathers(0, 0)
    wait_scatter(2, 0)
    process(0, 0)
    wait_scatter(3, 1)
    wait_scatter(0, 0)

    pltpu.sync_copy(deg_loc, deg_out.at[pl.ds(wid * NP, NP)])
    plsc.subcore_barrier()

    pltpu.sync_copy(agg_sh.at[pl.ds(s * RPS, RPS)],
                    agg_out.at[pl.ds(c * NP + s * RPS, RPS)])


def _sc_nodeagg(ms, md, src3, dst3):
    mesh = plsc.VectorSubcoreMesh(core_axis_name="c", subcore_axis_name="s",
                                  num_cores=NC, num_subcores=NS)
    zeros_agg = jnp.zeros((NP, D), _f32)
    fn = functools.partial(
        pl.kernel,
        out_type=[
            jax.ShapeDtypeStruct((NC * NP, D), _f32),
            jax.ShapeDtypeStruct((NW * NP,), _f32),
        ],
        mesh=mesh,
        scratch_types=[
            pltpu.VMEM((4, C1), jnp.int32),
            pltpu.VMEM((4, C1), jnp.int32),
            pltpu.VMEM((2, C1, D), _f32),
            pltpu.VMEM((2, C1, D), _f32),
            pltpu.VMEM((2, C1, D), _f32),
            pltpu.VMEM((NP,), _f32),
            pltpu.VMEM_SHARED((NP, D), _f32),
            pltpu.SemaphoreType.DMA,
            pltpu.SemaphoreType.DMA,
            pltpu.SemaphoreType.DMA,
            pltpu.SemaphoreType.DMA,
            pltpu.SemaphoreType.DMA,
        ],
        compiler_params=pltpu.CompilerParams(needs_layout_passes=False),
    )(_sc_nodeagg_body)
    return fn(ms, md, src3, dst3, zeros_agg)


def _tc_degsum_body(dp_ref, out_ref):
    out_ref[...] = jnp.sum(dp_ref[...], axis=0, keepdims=True)


def _tc_degsum(degp):
    bn = 2048
    return pl.pallas_call(
        _tc_degsum_body,
        grid=(NP // bn,),
        in_specs=[pl.BlockSpec((NW, bn), lambda i: (0, i))],
        out_specs=pl.BlockSpec((1, bn), lambda i: (0, i)),
        out_shape=jax.ShapeDtypeStruct((1, NP), _f32),
    )(degp.reshape(NW, NP))


def _tc_mid_body(x_ref, a0_ref, a1_ref, d_ref, wn_ref, bn_ref,
                 we1_ref, be1_ref, a_out, b_out):
    agg = a0_ref[...] + a1_ref[...]
    aggn = agg / jnp.maximum(d_ref[...], 1.0)
    x2 = jnp.maximum(
        jnp.dot(x_ref[...], wn_ref[0:D, :], preferred_element_type=_f32)
        + jnp.dot(aggn, wn_ref[D:2 * D, :], preferred_element_type=_f32)
        + bn_ref[...], 0.0)
    a_out[...] = jnp.dot(x2, we1_ref[0:D, :], preferred_element_type=_f32)
    b_out[...] = (jnp.dot(x2, we1_ref[D:2 * D, :], preferred_element_type=_f32)
                  + be1_ref[...])


def _tc_mid(x, agg0, agg1, deg, w_nout, b_nout, w_e1, b_e1):
    bn = 2000
    return pl.pallas_call(
        _tc_mid_body,
        grid=(N // bn,),
        in_specs=[
            pl.BlockSpec((bn, D), lambda i: (i, 0)),
            pl.BlockSpec((bn, D), lambda i: (i, 0)),
            pl.BlockSpec((bn, D), lambda i: (i, 0)),
            pl.BlockSpec((bn, 1), lambda i: (i, 0)),
            pl.BlockSpec((2 * D, D), lambda i: (0, 0)),
            pl.BlockSpec((1, D), lambda i: (0, 0)),
            pl.BlockSpec((2 * D, D), lambda i: (0, 0)),
            pl.BlockSpec((1, D), lambda i: (0, 0)),
        ],
        out_specs=[
            pl.BlockSpec((bn, D), lambda i: (i, 0)),
            pl.BlockSpec((bn, D), lambda i: (i, 0)),
        ],
        out_shape=[
            jax.ShapeDtypeStruct((N, D), _f32),
            jax.ShapeDtypeStruct((N, D), _f32),
        ],
    )(x, agg0, agg1, deg, w_nout, b_nout.reshape(1, D),
      w_e1, b_e1.reshape(1, D))


def _sc_edge_body(a_hbm, b_hbm, src_hbm, dst_hbm, w2_hbm,
                  dot_out, ssq_out,
                  sidx_all, didx_all, ga_v, gb_v, dot_v, w2_v, ssq_v,
                  sem_g0, sem_g1, sem_w0, sem_w1):
    c = lax.axis_index("c")
    s = lax.axis_index("s")
    wid = c * NS + s
    base = wid * EPT
    sem_g = (sem_g0, sem_g1)
    sem_w = (sem_w0, sem_w1)

    pltpu.sync_copy(w2_hbm, w2_v)
    pltpu.sync_copy(src_hbm.at[wid], sidx_all)
    pltpu.sync_copy(dst_hbm.at[wid], didx_all)

    def fire_gathers(k, b):
        pltpu.async_copy(a_hbm.at[sidx_all.at[k]], ga_v.at[b], sem_g[b])
        pltpu.async_copy(b_hbm.at[didx_all.at[k]], gb_v.at[b], sem_g[b])

    def wait_gathers(k, b):
        pltpu.make_async_copy(a_hbm.at[sidx_all.at[k]], ga_v.at[b],
                              sem_g[b]).wait()
        pltpu.make_async_copy(b_hbm.at[didx_all.at[k]], gb_v.at[b],
                              sem_g[b]).wait()

    def wait_write(k, b):
        pltpu.make_async_copy(dot_v.at[b],
                              dot_out.at[pl.ds(base + k * C2, C2)],
                              sem_w[b]).wait()

    def process(k, b, sacc0):
        def edge_body(e, sacc):
            dot = jnp.zeros((L,), _f32)
            for j in range(D // L):
                sl = pl.ds(j * L, L)
                pair = ga_v[b, e, sl] + gb_v[b, e, sl]
                sacc = sacc + pair * pair
                dot = dot + jnp.maximum(pair, 0.0) * w2_v[j, :]
            dot_v[b, e, :] = dot
            return sacc

        sacc0 = plsc.parallel_loop(0, C2, unroll=2, carry=sacc0)(edge_body)
        pltpu.async_copy(dot_v.at[b], dot_out.at[pl.ds(base + k * C2, C2)],
                         sem_w[b])
        return sacc0

    fire_gathers(0, 0)

    def loop_body(i, ssq_acc):
        for b in range(2):
            k = i * 2 + b
            wait_gathers(k, b)
            fire_gathers(k + 1, 1 - b)

            @pl.when(i >= 1)
            def _():
                wait_write(k - 2, b)
            ssq_acc = process(k, b, ssq_acc)
        return ssq_acc

    ssq = lax.fori_loop(0, (NCHUNK2 - 1) // 2, loop_body,
                        jnp.zeros((L,), _f32))
    klast = NCHUNK2 - 1
    wait_gathers(klast, 0)
    wait_write(klast - 2, 0)
    ssq = process(klast, 0, ssq)
    wait_write(klast, 0)
    wait_write(klast - 1, 1)

    for r in range(8):
        ssq_v[r, :] = jnp.zeros((L,), _f32)
    ssq_v[0, :] = ssq
    pltpu.sync_copy(ssq_v, ssq_out.at[wid])


def _sc_edge(a, b, src3, dst3, w2):
    mesh = plsc.VectorSubcoreMesh(core_axis_name="c", subcore_axis_name="s",
                                  num_cores=NC, num_subcores=NS)
    fn = functools.partial(
        pl.kernel,
        out_type=[
            jax.ShapeDtypeStruct((E, L), _f32),
            jax.ShapeDtypeStruct((NW, 8, L), _f32),
        ],
        mesh=mesh,
        scratch_types=[
            pltpu.VMEM((NCHUNK2, C2), jnp.int32),
            pltpu.VMEM((NCHUNK2, C2), jnp.int32),
            pltpu.VMEM((2, C2, D), _f32),
            pltpu.VMEM((2, C2, D), _f32),
            pltpu.VMEM((2, C2, L), _f32),
            pltpu.VMEM((D // L, L), _f32),
            pltpu.VMEM((8, L), _f32),
            pltpu.SemaphoreType.DMA,
            pltpu.SemaphoreType.DMA,
            pltpu.SemaphoreType.DMA,
            pltpu.SemaphoreType.DMA,
        ],
        compiler_params=pltpu.CompilerParams(needs_layout_passes=False),
    )(_sc_edge_body)
    return fn(a, b, src3, dst3, w2.reshape(D // L, L))


def _tc_final_body(dot16_ref, ef_ref, wtail_ref, be2_ref, ssq_ref,
                   out_ref, loss_ref):
    s = (jnp.sum(dot16_ref[...], axis=1, keepdims=True)
         + jnp.sum(ef_ref[...] * wtail_ref[...], axis=1, keepdims=True)
         + be2_ref[...])
    out_ref[...] = s

    @pl.when(pl.program_id(0) == 0)
    def _():
        loss_ref[...] = jnp.sum(ssq_ref[...]).reshape(1, 1) / (E * D)


def _tc_final(dot16, ef, w_tail, b_e2, ssq):
    be = 8000
    return pl.pallas_call(
        _tc_final_body,
        grid=(E // be,),
        in_specs=[
            pl.BlockSpec((be, L), lambda i: (i, 0)),
            pl.BlockSpec((be, DE), lambda i: (i, 0)),
            pl.BlockSpec((1, DE), lambda i: (0, 0)),
            pl.BlockSpec((1, 1), lambda i: (0, 0)),
            pl.BlockSpec((NW, 8, L), lambda i: (0, 0, 0)),
        ],
        out_specs=[
            pl.BlockSpec((be, 1), lambda i: (i, 0)),
            pl.BlockSpec((1, 1), lambda i: (0, 0)),
        ],
        out_shape=[
            jax.ShapeDtypeStruct((E, 1), _f32),
            jax.ShapeDtypeStruct((1, 1), _f32),
        ],
    )(dot16, ef, w_tail, b_e2, ssq)


def kernel(node_features, edge_features, edge_index, gt_edges,
           W_msg, b_msg, W_nout, b_nout, W_e1, b_e1, W_e2, b_e2):
    pad = jnp.full((EP - E,), NP - 1, edge_index.dtype)
    src1 = jnp.concatenate([edge_index[0], pad]).reshape(NW, NCHUNK1, C1)
    dst1 = jnp.concatenate([edge_index[1], pad]).reshape(NW, NCHUNK1, C1)
    src2 = edge_index[0].reshape(NW, NCHUNK2, C2)
    dst2 = edge_index[1].reshape(NW, NCHUNK2, C2)

    ms, md = _tc_prep(node_features, W_msg, b_msg)
    aggp, degp = _sc_nodeagg(ms, md, src1, dst1)
    deg = _tc_degsum(degp).reshape(NP)[:N].reshape(N, 1)
    a, b = _tc_mid(node_features, aggp[:N], aggp[NP:NP + N], deg,
                   W_nout, b_nout, W_e1, b_e1)
    dot16, ssq = _sc_edge(a, b, src2, dst2, W_e2[:D, 0])
    w_tail = W_e2[D:D + DE, 0].reshape(1, DE)
    edge_out, loss = _tc_final(dot16, edge_features, w_tail,
                               b_e2.reshape(1, 1), ssq)
    return edge_out, loss.reshape(())


# speedup vs baseline: 5.3722x; 1.0132x over previous
import functools

import jax
import jax.numpy as jnp
from jax import lax
from jax.experimental import pallas as pl
from jax.experimental.pallas import tpu as pltpu
from jax.experimental.pallas import tpu_sc as plsc

N = 10000
E = 320000
D = 128
DE = 16
NC, NS, L = 2, 16, 16
NW = NC * NS
EPT = E // NW
EP = 320512
EPT1 = EP // NW
C1 = 32
NCHUNK1 = EPT1 // C1
NITER1 = NCHUNK1 // 4
C2 = 80
NCHUNK2 = EPT // C2
NP = 10240
RPS = NP // NS

_f32 = jnp.float32


def _tc_prep_body(x_ref, wm_ref, bm_ref, ms_ref, md_ref):
    x = x_ref[...]
    ms_ref[...] = jnp.dot(x, wm_ref[0:D, :], preferred_element_type=_f32)
    md_ref[...] = (jnp.dot(x, wm_ref[D:2 * D, :], preferred_element_type=_f32)
                   + bm_ref[...])


def _tc_prep(xp, w_msg, b_msg):
    bn = 2048
    return pl.pallas_call(
        _tc_prep_body,
        grid=(NP // bn,),
        in_specs=[
            pl.BlockSpec((bn, D), lambda i: (i, 0)),
            pl.BlockSpec((2 * D, D), lambda i: (0, 0)),
            pl.BlockSpec((1, D), lambda i: (0, 0)),
        ],
        out_specs=[
            pl.BlockSpec((bn, D), lambda i: (i, 0)),
            pl.BlockSpec((bn, D), lambda i: (i, 0)),
        ],
        out_shape=[
            jax.ShapeDtypeStruct((NP, D), _f32),
            jax.ShapeDtypeStruct((NP, D), _f32),
        ],
    )(xp, w_msg, b_msg.reshape(1, D))


def _sc_nodeagg_body(ms_hbm, md_hbm, src_hbm, dst_hbm, zeros_agg_hbm,
                     agg_out, deg_out,
                     sidx_v, didx_v, gs_v, gd_v, m_v, deg_loc,
                     agg_sh, sem_i, sem_g0, sem_g1, sem_s0, sem_s1):
    c = lax.axis_index("c")
    s = lax.axis_index("s")
    wid = c * NS + s
    sem_g = (sem_g0, sem_g1)
    sem_s = (sem_s0, sem_s1)

    pltpu.sync_copy(zeros_agg_hbm.at[pl.ds(s * RPS, RPS)],
                    agg_sh.at[pl.ds(s * RPS, RPS)])

    def zero_body(r, _):
        deg_loc[pl.ds(r * L, L)] = jnp.zeros((L,), _f32)
        return 0
    lax.fori_loop(0, NP // L, zero_body, 0)
    plsc.subcore_barrier()

    ones16 = jnp.full((L,), 1.0, _f32)

    def fire_idx(k, slot):
        pltpu.async_copy(src_hbm.at[wid, k], sidx_v.at[slot], sem_i)
        pltpu.async_copy(dst_hbm.at[wid, k], didx_v.at[slot], sem_i)

    def wait_idx(k, slot):
        pltpu.make_async_copy(src_hbm.at[wid, k], sidx_v.at[slot],
                              sem_i).wait()
        pltpu.make_async_copy(dst_hbm.at[wid, k], didx_v.at[slot],
                              sem_i).wait()

    def fire_gathers(slot, b):
        pltpu.async_copy(ms_hbm.at[sidx_v.at[slot]], gs_v.at[b], sem_g[b])
        pltpu.async_copy(md_hbm.at[didx_v.at[slot]], gd_v.at[b], sem_g[b])

    def wait_gathers(slot, b):
        pltpu.make_async_copy(ms_hbm.at[sidx_v.at[slot]], gs_v.at[b],
                              sem_g[b]).wait()
        pltpu.make_async_copy(md_hbm.at[didx_v.at[slot]], gd_v.at[b],
                              sem_g[b]).wait()

    def wait_scatter(slot, b):
        pltpu.make_async_copy(m_v.at[b], agg_sh.at[didx_v.at[slot]],
                              sem_s[b]).wait()

    def process(slot, b):
        @plsc.parallel_loop(0, C1, unroll=4)
        def _(e):
            for j in range(D // L):
                sl = pl.ds(j * L, L)
                m_v[b, e, sl] = jnp.maximum(
                    gs_v[b, e, sl] + gd_v[b, e, sl], 0.0)
        for g in range(C1 // L):
            plsc.addupdate_scatter(
                deg_loc, [didx_v[slot, pl.ds(g * L, L)]], ones16)
        pltpu.async_copy(m_v.at[b], agg_sh.at[didx_v.at[slot]], sem_s[b],
                         add=True)

    fire_idx(0, 0)
    fire_idx(1, 1)
    wait_idx(0, 0)
    fire_gathers(0, 0)

    def loop_body(i, carry):
        for b in range(4):
            k = i * 4 + b
            gb = b % 2
            wait_gathers(b, gb)
            wait_idx(k + 1, (b + 1) % 4)
            fire_gathers((b + 1) % 4, 1 - gb)
            if b < 2:
                @pl.when(i >= 1)
                def _():
                    wait_scatter((b + 2) % 4, gb)
            else:
                wait_scatter((b + 2) % 4, gb)
            if b < 3:
                fire_idx(k + 2, (b + 2) % 4)
            else:
                @pl.when(i < NITER1 - 1)
                def _():
                    fire_idx(k + 2, (b + 2) % 4)
            process(b, gb)
        return carry

    lax.fori_loop(0, NITER1, loop_body, 0)
    wait_gathers(0, 0)
    wait_scatter(2, 0)
    process(0, 0)
    wait_scatter(3, 1)
    wait_scatter(0, 0)

    pltpu.sync_copy(deg_loc, deg_out.at[pl.ds(wid * NP, NP)])
    plsc.subcore_barrier()

    pltpu.sync_copy(agg_sh.at[pl.ds(s * RPS, RPS)],
                    agg_out.at[pl.ds(c * NP + s * RPS, RPS)])


def _sc_nodeagg(ms, md, src3, dst3):
    mesh = plsc.VectorSubcoreMesh(core_axis_name="c", subcore_axis_name="s",
                                  num_cores=NC, num_subcores=NS)
    zeros_agg = jnp.zeros((NP, D), _f32)
    fn = functools.partial(
        pl.kernel,
        out_type=[
            jax.ShapeDtypeStruct((NC * NP, D), _f32),
            jax.ShapeDtypeStruct((NW * NP,), _f32),
        ],
        mesh=mesh,
        scratch_types=[
            pltpu.VMEM((4, C1), jnp.int32),
            pltpu.VMEM((4, C1), jnp.int32),
            pltpu.VMEM((2, C1, D), _f32),
            pltpu.VMEM((2, C1, D), _f32),
            pltpu.VMEM((2, C1, D), _f32),
            pltpu.VMEM((NP,), _f32),
            pltpu.VMEM_SHARED((NP, D), _f32),
            pltpu.SemaphoreType.DMA,
            pltpu.SemaphoreType.DMA,
            pltpu.SemaphoreType.DMA,
            pltpu.SemaphoreType.DMA,
            pltpu.SemaphoreType.DMA,
        ],
        compiler_params=pltpu.CompilerParams(needs_layout_passes=False),
    )(_sc_nodeagg_body)
    return fn(ms, md, src3, dst3, zeros_agg)


def _tc_mid_body(x_ref, a0_ref, a1_ref, dp_ref, wn_ref, bn_ref,
                 we1_ref, be1_ref, a_out, b_out):
    agg = a0_ref[...] + a1_ref[...]
    deg = lax.dot_general(dp_ref[...], jnp.ones((NW, 1), _f32),
                          (((0,), (0,)), ((), ())),
                          preferred_element_type=_f32)
    aggn = agg / jnp.maximum(deg, 1.0)
    x2 = jnp.maximum(
        jnp.dot(x_ref[...], wn_ref[0:D, :], preferred_element_type=_f32)
        + jnp.dot(aggn, wn_ref[D:2 * D, :], preferred_element_type=_f32)
        + bn_ref[...], 0.0)
    a_out[...] = jnp.dot(x2, we1_ref[0:D, :], preferred_element_type=_f32)
    b_out[...] = (jnp.dot(x2, we1_ref[D:2 * D, :], preferred_element_type=_f32)
                  + be1_ref[...])


def _tc_mid(x, agg0, agg1, degp, w_nout, b_nout, w_e1, b_e1):
    bn = 2048
    return pl.pallas_call(
        _tc_mid_body,
        grid=(NP // bn,),
        in_specs=[
            pl.BlockSpec((bn, D), lambda i: (i, 0)),
            pl.BlockSpec((bn, D), lambda i: (i, 0)),
            pl.BlockSpec((bn, D), lambda i: (i, 0)),
            pl.BlockSpec((NW, bn), lambda i: (0, i)),
            pl.BlockSpec((2 * D, D), lambda i: (0, 0)),
            pl.BlockSpec((1, D), lambda i: (0, 0)),
            pl.BlockSpec((2 * D, D), lambda i: (0, 0)),
            pl.BlockSpec((1, D), lambda i: (0, 0)),
        ],
        out_specs=[
            pl.BlockSpec((bn, D), lambda i: (i, 0)),
            pl.BlockSpec((bn, D), lambda i: (i, 0)),
        ],
        out_shape=[
            jax.ShapeDtypeStruct((NP, D), _f32),
            jax.ShapeDtypeStruct((NP, D), _f32),
        ],
    )(x, agg0, agg1, degp, w_nout, b_nout.reshape(1, D),
      w_e1, b_e1.reshape(1, D))


def _sc_edge_body(a_hbm, b_hbm, src_hbm, dst_hbm, w2_hbm,
                  dot_out, ssq_out,
                  sidx_all, didx_all, ga_v, gb_v, dot_v, w2_v, ssq_v,
                  sem_g0, sem_g1, sem_w0, sem_w1):
    c = lax.axis_index("c")
    s = lax.axis_index("s")
    wid = c * NS + s
    base = wid * EPT
    sem_g = (sem_g0, sem_g1)
    sem_w = (sem_w0, sem_w1)

    pltpu.sync_copy(w2_hbm, w2_v)
    pltpu.sync_copy(src_hbm.at[wid], sidx_all)
    pltpu.sync_copy(dst_hbm.at[wid], didx_all)

    def fire_gathers(k, b):
        pltpu.async_copy(a_hbm.at[sidx_all.at[k]], ga_v.at[b], sem_g[b])
        pltpu.async_copy(b_hbm.at[didx_all.at[k]], gb_v.at[b], sem_g[b])

    def wait_gathers(k, b):
        pltpu.make_async_copy(a_hbm.at[sidx_all.at[k]], ga_v.at[b],
                              sem_g[b]).wait()
        pltpu.make_async_copy(b_hbm.at[didx_all.at[k]], gb_v.at[b],
                              sem_g[b]).wait()

    def wait_write(k, b):
        pltpu.make_async_copy(dot_v.at[b],
                              dot_out.at[pl.ds(base + k * C2, C2)],
                              sem_w[b]).wait()

    def process(k, b, sacc0):
        def edge_body(e, sacc):
            dot = jnp.zeros((L,), _f32)
            for j in range(D // L):
                sl = pl.ds(j * L, L)
                pair = ga_v[b, e, sl] + gb_v[b, e, sl]
                sacc = sacc + pair * pair
                dot = dot + jnp.maximum(pair, 0.0) * w2_v[j, :]
            dot_v[b, e, :] = dot
            return sacc

        sacc0 = plsc.parallel_loop(0, C2, unroll=2, carry=sacc0)(edge_body)
        pltpu.async_copy(dot_v.at[b], dot_out.at[pl.ds(base + k * C2, C2)],
                         sem_w[b])
        return sacc0

    fire_gathers(0, 0)

    def loop_body(i, ssq_acc):
        for b in range(2):
            k = i * 2 + b
            wait_gathers(k, b)
            fire_gathers(k + 1, 1 - b)

            @pl.when(i >= 1)
            def _():
                wait_write(k - 2, b)
            ssq_acc = process(k, b, ssq_acc)
        return ssq_acc

    ssq = lax.fori_loop(0, (NCHUNK2 - 1) // 2, loop_body,
                        jnp.zeros((L,), _f32))
    klast = NCHUNK2 - 1
    wait_gathers(klast, 0)
    wait_write(klast - 2, 0)
    ssq = process(klast, 0, ssq)
    wait_write(klast, 0)
    wait_write(klast - 1, 1)

    for r in range(8):
        ssq_v[r, :] = jnp.zeros((L,), _f32)
    ssq_v[0, :] = ssq
    pltpu.sync_copy(ssq_v, ssq_out.at[wid])


def _sc_edge(a, b, src3, dst3, w2):
    mesh = plsc.VectorSubcoreMesh(core_axis_name="c", subcore_axis_name="s",
                                  num_cores=NC, num_subcores=NS)
    fn = functools.partial(
        pl.kernel,
        out_type=[
            jax.ShapeDtypeStruct((E, L), _f32),
            jax.ShapeDtypeStruct((NW, 8, L), _f32),
        ],
        mesh=mesh,
        scratch_types=[
            pltpu.VMEM((NCHUNK2, C2), jnp.int32),
            pltpu.VMEM((NCHUNK2, C2), jnp.int32),
            pltpu.VMEM((2, C2, D), _f32),
            pltpu.VMEM((2, C2, D), _f32),
            pltpu.VMEM((2, C2, L), _f32),
            pltpu.VMEM((D // L, L), _f32),
            pltpu.VMEM((8, L), _f32),
            pltpu.SemaphoreType.DMA,
            pltpu.SemaphoreType.DMA,
            pltpu.SemaphoreType.DMA,
            pltpu.SemaphoreType.DMA,
        ],
        compiler_params=pltpu.CompilerParams(needs_layout_passes=False),
    )(_sc_edge_body)
    return fn(a, b, src3, dst3, w2.reshape(D // L, L))


def _tc_final_body(dot16_ref, ef_ref, wtail_ref, be2_ref, ssq_ref,
                   out_ref, loss_ref):
    s = (jnp.sum(dot16_ref[...], axis=1, keepdims=True)
         + jnp.sum(ef_ref[...] * wtail_ref[...], axis=1, keepdims=True)
         + be2_ref[...])
    out_ref[...] = s

    @pl.when(pl.program_id(0) == 0)
    def _():
        loss_ref[...] = jnp.sum(ssq_ref[...]).reshape(1, 1) / (E * D)


def _tc_final(dot16, ef, w_tail, b_e2, ssq):
    be = 8000
    return pl.pallas_call(
        _tc_final_body,
        grid=(E // be,),
        in_specs=[
            pl.BlockSpec((be, L), lambda i: (i, 0)),
            pl.BlockSpec((be, DE), lambda i: (i, 0)),
            pl.BlockSpec((1, DE), lambda i: (0, 0)),
            pl.BlockSpec((1, 1), lambda i: (0, 0)),
            pl.BlockSpec((NW, 8, L), lambda i: (0, 0, 0)),
        ],
        out_specs=[
            pl.BlockSpec((be, 1), lambda i: (i, 0)),
            pl.BlockSpec((1, 1), lambda i: (0, 0)),
        ],
        out_shape=[
            jax.ShapeDtypeStruct((E, 1), _f32),
            jax.ShapeDtypeStruct((1, 1), _f32),
        ],
    )(dot16, ef, w_tail, b_e2, ssq)


def kernel(node_features, edge_features, edge_index, gt_edges,
           W_msg, b_msg, W_nout, b_nout, W_e1, b_e1, W_e2, b_e2):
    pad = jnp.full((EP - E,), NP - 1, edge_index.dtype)
    src1 = jnp.concatenate([edge_index[0], pad]).reshape(NW, NCHUNK1, C1)
    dst1 = jnp.concatenate([edge_index[1], pad]).reshape(NW, NCHUNK1, C1)
    src2 = edge_index[0].reshape(NW, NCHUNK2, C2)
    dst2 = edge_index[1].reshape(NW, NCHUNK2, C2)

    xp = jnp.concatenate([node_features, jnp.zeros((NP - N, D), _f32)])
    ms, md = _tc_prep(xp, W_msg, b_msg)
    aggp, degp = _sc_nodeagg(ms, md, src1, dst1)
    a, b = _tc_mid(xp, aggp[:NP], aggp[NP:2 * NP],
                   degp.reshape(NW, NP), W_nout, b_nout, W_e1, b_e1)
    dot16, ssq = _sc_edge(a, b, src2, dst2, W_e2[:D, 0])
    w_tail = W_e2[D:D + DE, 0].reshape(1, DE)
    edge_out, loss = _tc_final(dot16, edge_features, w_tail,
                               b_e2.reshape(1, 1), ssq)
    return edge_out, loss.reshape(())
